# baseline (speedup 1.0000x reference)
import functools

import jax
import jax.numpy as jnp
from jax import lax
from jax.experimental import pallas as pl
from jax.experimental.pallas import tpu as pltpu
from jax.experimental.pallas import tpu_sc as plsc

_BETA = 0.2
_L = 16
_NC = 2
_NS = 16
_NW = _NC * _NS
_K = 128


def _sc_update(M, D, N, mem, idx, val, out,
               idx_v, winner_v, widx_v, wj_v, stage_v, rows_a, rows_b,
               bidx_v, brow_v,
               sem_copy, sem_a, sem_b):
    R = M // _NW
    RLIN = (R - 8) // 8 * 8
    NB = R - RLIN
    wid = lax.axis_index("s") * _NC + lax.axis_index("c")
    base = wid * R
    lane = lax.iota(jnp.int32, _L)

    hn = (8 - base % 8) % 8
    b_lin = pl.multiple_of(base + hn, 8)
    cdesc = pltpu.async_copy(mem.at[pl.ds(b_lin, RLIN)],
                             out.at[pl.ds(b_lin, RLIN)], sem_copy)

    bl = jnp.where(lane < hn, base + lane, b_lin + RLIN + (lane - hn))
    bl = jnp.where(lane < NB, bl, base)
    bidx_v[0, :] = bl
    pltpu.async_copy(mem.at[bidx_v.at[0]], brow_v, sem_a).wait()
    pltpu.sync_copy(brow_v, out.at[bidx_v.at[0]])

    pltpu.sync_copy(idx, idx_v)

    nch = N // _L

    def p1(c, carry):
        jv = idx_v[pl.ds(c * _L, _L)]
        pos = lane + c * _L
        loc = jv - base
        m = (loc >= 0) & (loc < R)
        locc = jnp.where(m, loc, 0)
        plsc.store_scatter(winner_v, [locc], pos, mask=m)

        def fix_body(n):
            rb = plsc.load_gather(winner_v, [locc], mask=m)
            need = m & (rb < pos)
            plsc.store_scatter(winner_v, [locc], pos, mask=need)
            return jnp.sum(need.astype(jnp.int32))

        lax.while_loop(lambda n: n > 0, fix_body, jnp.int32(1))
        return carry

    lax.fori_loop(0, nch, p1, jnp.int32(0))

    def p2(c, cnt):
        jv = idx_v[pl.ds(c * _L, _L)]
        pos = lane + c * _L
        loc = jv - base
        m = (loc >= 0) & (loc < R)
        locc = jnp.where(m, loc, 0)
        rb = plsc.load_gather(winner_v, [locc], mask=m)
        win = m & (rb == pos)
        plsc.store_compressed(widx_v.at[pl.ds(cnt, _L)], jv, mask=win)
        plsc.store_compressed(wj_v.at[pl.ds(cnt, _L)], pos, mask=win)
        return cnt + jnp.sum(win.astype(jnp.int32))

    cnt = lax.fori_loop(0, nch, p2, jnp.int32(0))

    lastp = jnp.full((_L,), 0, jnp.int32) + jnp.maximum(cnt - 1, 0)
    lastw = plsc.load_gather(widx_v, [lastp])
    lastj = plsc.load_gather(wj_v, [lastp])

    def pad(q, carry):
        widx_v[pl.ds(cnt + q * _L, _L)] = lastw
        wj_v[pl.ds(cnt + q * _L, _L)] = lastj
        return carry

    lax.fori_loop(0, _K // _L, pad, jnp.int32(0))

    cdesc.wait()
    nt = (cnt + _K - 1) // _K

    def p3(t, carry):
        off = t * _K

        def st(q, c2):
            stage_v[0, pl.ds(q * _L, _L)] = widx_v[pl.ds(off + q * _L, _L)]
            return c2

        lax.fori_loop(0, _K // _L, st, jnp.int32(0), unroll=True)

        ga = pltpu.async_copy(mem.at[widx_v.at[pl.ds(off, _K)]], rows_a, sem_a)
        gb = pltpu.async_copy(val.at[wj_v.at[pl.ds(off, _K)]], rows_b, sem_b)
        ga.wait()
        gb.wait()

        def blend(r, c2):
            for q in range(D // _L):
                s = pl.ds(q * _L, _L)
                rows_a[r, s] = (rows_a[r, s] * _BETA
                                + rows_b[r, s] * (1.0 - _BETA))
            return c2

        lax.fori_loop(0, _K, blend, jnp.int32(0))
        pltpu.sync_copy(rows_a, out.at[stage_v.at[0]])
        return carry

    lax.fori_loop(0, nt, p3, jnp.int32(0))


def kernel(mem, idx, val):
    M, D = mem.shape
    N = idx.shape[0]
    assert M % _NW == 0 and N % _L == 0 and D % _L == 0

    mesh = plsc.VectorSubcoreMesh(core_axis_name="c", subcore_axis_name="s")
    cap = N + _K + _L
    run = pl.kernel(
        functools.partial(_sc_update, M, D, N),
        out_type=jax.ShapeDtypeStruct((M, D), jnp.float32),
        mesh=mesh,
        compiler_params=pltpu.CompilerParams(use_tc_tiling_on_sc=False,
                                             needs_layout_passes=False),
        scratch_types=[
            pltpu.VMEM((N,), jnp.int32),
            pltpu.VMEM((M // _NW,), jnp.int32),
            pltpu.VMEM((cap,), jnp.int32),
            pltpu.VMEM((cap,), jnp.int32),
            pltpu.VMEM((1, _K), jnp.int32),
            pltpu.VMEM((_K, D), jnp.float32),
            pltpu.VMEM((_K, D), jnp.float32),
            pltpu.VMEM((1, _L), jnp.int32),
            pltpu.VMEM((_L, D), jnp.float32),
            pltpu.SemaphoreType.DMA,
            pltpu.SemaphoreType.DMA,
            pltpu.SemaphoreType.DMA,
        ],
    )
    return run(mem, idx.astype(jnp.int32), val)


# speedup vs baseline: 5.1468x; 5.1468x over previous
import functools

import jax
import jax.numpy as jnp
from jax import lax
from jax.experimental import pallas as pl
from jax.experimental.pallas import tpu as pltpu
from jax.experimental.pallas import tpu_sc as plsc

_BETA = 0.2
_L = 16
_NC = 2
_NS = 16
_NW = _NC * _NS
_K = 128
_BR = 8000


def _copy_body(x_ref, o_ref):
    o_ref[...] = x_ref[...]


def _tc_copy(mem):
    M, D = mem.shape
    return pl.pallas_call(
        _copy_body,
        grid=(M // _BR,),
        in_specs=[pl.BlockSpec((_BR, D), lambda i: (i, 0))],
        out_specs=pl.BlockSpec((_BR, D), lambda i: (i, 0)),
        out_shape=jax.ShapeDtypeStruct((M, D), jnp.float32),
    )(mem)


def _sc_update(M, D, N, idx, val, y,
               idx_v, winner_v, widx_v, wj_v, stage_v, rows_a, rows_b,
               sem_a, sem_b):
    R = M // _NW
    wid = lax.axis_index("s") * _NC + lax.axis_index("c")
    base = wid * R
    lane = lax.iota(jnp.int32, _L)

    pltpu.sync_copy(idx, idx_v)

    nch = N // _L

    def p1(c, carry):
        jv = idx_v[pl.ds(c * _L, _L)]
        pos = lane + c * _L
        loc = jv - base
        m = (loc >= 0) & (loc < R)
        locc = jnp.where(m, loc, 0)
        plsc.store_scatter(winner_v, [locc], pos, mask=m)

        def fix_body(n):
            rb = plsc.load_gather(winner_v, [locc], mask=m)
            need = m & (rb < pos)
            plsc.store_scatter(winner_v, [locc], pos, mask=need)
            return jnp.sum(need.astype(jnp.int32))

        lax.while_loop(lambda n: n > 0, fix_body, jnp.int32(1))
        return carry

    lax.fori_loop(0, nch, p1, jnp.int32(0))

    def p2(c, cnt):
        jv = idx_v[pl.ds(c * _L, _L)]
        pos = lane + c * _L
        loc = jv - base
        m = (loc >= 0) & (loc < R)
        locc = jnp.where(m, loc, 0)
        rb = plsc.load_gather(winner_v, [locc], mask=m)
        win = m & (rb == pos)
        plsc.store_compressed(widx_v.at[pl.ds(cnt, _L)], jv, mask=win)
        plsc.store_compressed(wj_v.at[pl.ds(cnt, _L)], pos, mask=win)
        return cnt + jnp.sum(win.astype(jnp.int32))

    cnt = lax.fori_loop(0, nch, p2, jnp.int32(0))

    lastp = jnp.full((_L,), 0, jnp.int32) + jnp.maximum(cnt - 1, 0)
    lastw = plsc.load_gather(widx_v, [lastp])
    lastj = plsc.load_gather(wj_v, [lastp])

    def pad(q, carry):
        widx_v[pl.ds(cnt + q * _L, _L)] = lastw
        wj_v[pl.ds(cnt + q * _L, _L)] = lastj
        return carry

    lax.fori_loop(0, _K // _L, pad, jnp.int32(0))

    nt = (cnt + _K - 1) // _K

    def p3(t, carry):
        off = t * _K

        def st(q, c2):
            stage_v[0, pl.ds(q * _L, _L)] = widx_v[pl.ds(off + q * _L, _L)]
            return c2

        lax.fori_loop(0, _K // _L, st, jnp.int32(0), unroll=True)

        ga = pltpu.async_copy(y.at[widx_v.at[pl.ds(off, _K)]], rows_a, sem_a)
        gb = pltpu.async_copy(val.at[wj_v.at[pl.ds(off, _K)]], rows_b, sem_b)
        ga.wait()
        gb.wait()

        def blend(r, c2):
            for q in range(D // _L):
                s = pl.ds(q * _L, _L)
                rows_a[r, s] = (rows_a[r, s] * _BETA
                                + rows_b[r, s] * (1.0 - _BETA))
            return c2

        lax.fori_loop(0, _K, blend, jnp.int32(0))
        pltpu.sync_copy(rows_a, y.at[stage_v.at[0]])
        return carry

    lax.fori_loop(0, nt, p3, jnp.int32(0))


def kernel(mem, idx, val):
    M, D = mem.shape
    N = idx.shape[0]
    assert M % (_NW * 2) == 0 and N % _L == 0 and D % _L == 0 and M % _BR == 0

    mesh = plsc.VectorSubcoreMesh(core_axis_name="c", subcore_axis_name="s")
    cap = N + _K + _L
    run = pl.kernel(
        functools.partial(_sc_update, M, D, N),
        out_type=(),
        mesh=mesh,
        compiler_params=pltpu.CompilerParams(use_tc_tiling_on_sc=False,
                                             needs_layout_passes=False),
        scratch_types=[
            pltpu.VMEM((N,), jnp.int32),
            pltpu.VMEM((M // _NW,), jnp.int32),
            pltpu.VMEM((cap,), jnp.int32),
            pltpu.VMEM((cap,), jnp.int32),
            pltpu.VMEM((1, _K), jnp.int32),
            pltpu.VMEM((_K, D), jnp.float32),
            pltpu.VMEM((_K, D), jnp.float32),
            pltpu.SemaphoreType.DMA,
            pltpu.SemaphoreType.DMA,
        ],
    )
    y = _tc_copy(mem)
    y_ref = jax.new_ref(y)
    run(idx.astype(jnp.int32), val, y_ref)
    return y_ref[...]


# speedup vs baseline: 6.2379x; 1.2120x over previous
import functools

import jax
import jax.numpy as jnp
from jax import lax
from jax.experimental import pallas as pl
from jax.experimental.pallas import tpu as pltpu
from jax.experimental.pallas import tpu_sc as plsc

_BETA = 0.2
_L = 16
_NC = 2
_NS = 16
_NW = _NC * _NS
_K = 128
_BR = 5000


def _copy_body(x_ref, o_ref):
    o_ref[...] = x_ref[...]


def _tc_copy(mem):
    M, D = mem.shape
    rows = M * D // 128
    x = mem.reshape(rows, 128)
    y = pl.pallas_call(
        _copy_body,
        grid=(rows // _BR,),
        in_specs=[pl.BlockSpec((_BR, 128), lambda i: (i, 0))],
        out_specs=pl.BlockSpec((_BR, 128), lambda i: (i, 0)),
        out_shape=jax.ShapeDtypeStruct((rows, 128), jnp.float32),
    )(x)
    return y.reshape(M, D)


def _sc_update(M, D, N, idx, val, y,
               idx_v, winner_v, widx_v, wj_v, stage_v, rows_a, rows_b,
               sem_a, sem_b):
    R = M // _NW
    wid = lax.axis_index("s") * _NC + lax.axis_index("c")
    base = wid * R
    lane = lax.iota(jnp.int32, _L)

    pltpu.sync_copy(idx, idx_v)

    nch = N // _L

    def p1(c, carry):
        jv = idx_v[pl.ds(c * _L, _L)]
        pos = lane + c * _L
        loc = jv - base
        m = (loc >= 0) & (loc < R)
        locc = jnp.where(m, loc, 0)
        _, lastm = plsc.scan_count(jv, mask=m)
        plsc.store_scatter(winner_v, [locc], pos, mask=m & lastm)
        return carry

    lax.fori_loop(0, nch, p1, jnp.int32(0))

    def p2(c, cnt):
        jv = idx_v[pl.ds(c * _L, _L)]
        pos = lane + c * _L
        loc = jv - base
        m = (loc >= 0) & (loc < R)
        locc = jnp.where(m, loc, 0)
        rb = plsc.load_gather(winner_v, [locc], mask=m)
        win = m & (rb == pos)
        plsc.store_compressed(widx_v.at[pl.ds(cnt, _L)], jv, mask=win)
        plsc.store_compressed(wj_v.at[pl.ds(cnt, _L)], pos, mask=win)
        return cnt + jnp.sum(win.astype(jnp.int32))

    cnt = lax.fori_loop(0, nch, p2, jnp.int32(0))

    lastp = jnp.full((_L,), 0, jnp.int32) + jnp.maximum(cnt - 1, 0)
    lastw = plsc.load_gather(widx_v, [lastp])
    lastj = plsc.load_gather(wj_v, [lastp])

    def pad(q, carry):
        widx_v[pl.ds(cnt + q * _L, _L)] = lastw
        wj_v[pl.ds(cnt + q * _L, _L)] = lastj
        return carry

    lax.fori_loop(0, _K // _L, pad, jnp.int32(0))

    nt = (cnt + _K - 1) // _K

    def p3(t, carry):
        off = t * _K

        def st(q, c2):
            stage_v[0, pl.ds(q * _L, _L)] = widx_v[pl.ds(off + q * _L, _L)]
            return c2

        lax.fori_loop(0, _K // _L, st, jnp.int32(0), unroll=True)

        ga = pltpu.async_copy(y.at[widx_v.at[pl.ds(off, _K)]], rows_a, sem_a)
        gb = pltpu.async_copy(val.at[wj_v.at[pl.ds(off, _K)]], rows_b, sem_b)
        ga.wait()
        gb.wait()

        def blend(r, c2):
            for q in range(D // _L):
                s = pl.ds(q * _L, _L)
                rows_a[r, s] = (rows_a[r, s] * _BETA
                                + rows_b[r, s] * (1.0 - _BETA))
            return c2

        lax.fori_loop(0, _K, blend, jnp.int32(0))
        pltpu.sync_copy(rows_a, y.at[stage_v.at[0]])
        return carry

    lax.fori_loop(0, nt, p3, jnp.int32(0))


def kernel(mem, idx, val):
    M, D = mem.shape
    N = idx.shape[0]
    assert M % (_NW * 2) == 0 and N % _L == 0 and D % _L == 0
    assert (M * D // 128) % _BR == 0

    mesh = plsc.VectorSubcoreMesh(core_axis_name="c", subcore_axis_name="s")
    cap = N + _K + _L
    run = pl.kernel(
        functools.partial(_sc_update, M, D, N),
        out_type=(),
        mesh=mesh,
        compiler_params=pltpu.CompilerParams(use_tc_tiling_on_sc=False,
                                             needs_layout_passes=False),
        scratch_types=[
            pltpu.VMEM((N,), jnp.int32),
            pltpu.VMEM((M // _NW,), jnp.int32),
            pltpu.VMEM((cap,), jnp.int32),
            pltpu.VMEM((cap,), jnp.int32),
            pltpu.VMEM((1, _K), jnp.int32),
            pltpu.VMEM((_K, D), jnp.float32),
            pltpu.VMEM((_K, D), jnp.float32),
            pltpu.SemaphoreType.DMA,
            pltpu.SemaphoreType.DMA,
        ],
    )
    y = _tc_copy(mem)
    y_ref = jax.new_ref(y)
    run(idx.astype(jnp.int32), val, y_ref)
    return y_ref[...]
